# baseline (speedup 1.0000x reference)
import functools

import jax
import jax.numpy as jnp
from jax import lax
from jax.experimental import pallas as pl
from jax.experimental.pallas import tpu as pltpu
from jax.experimental.pallas import tpu_sc as plsc

NUM_CLASSES = 2
TOP_K = 750
NMS_THRESH = 0.3
CONF_THRESH = 0.05
NMS_TOP_K = 5000
NUM_PRIORS = 20000

_N = 20480
_NS = 16
_SH = _N // _NS
_CH = _SH // 16
_ROWS_PAD = 768
_RPT = _ROWS_PAD // _NS


_BIS0 = 0
_WIN0 = 32
_ARENA_ROWS = 64


def _sc_body(loc_h, sc_h, pr_h, out_h,
             l0, l1, l2, l3, p0, p1, p2, p3,
             x1r, y1r, x2r, y2r, arr, scv,
             wall_f, cnt_all, vtmp_f, rows_loc,
             arena):
    f32 = jnp.float32
    i32 = jnp.int32
    c = lax.axis_index("c")
    s = lax.axis_index("s")
    base = s * _SH
    i16 = lax.iota(i32, 16)

    def _zr(i, carry):
        rows_loc[i] = jnp.zeros((16,), f32)
        return carry

    lax.fori_loop(0, _ROWS_PAD, _zr, 0)

    for k, dst in enumerate((l0, l1, l2, l3)):
        pltpu.sync_copy(loc_h.at[c, k, pl.ds(base, _SH)], dst)
    for k, dst in enumerate((p0, p1, p2, p3)):
        pltpu.sync_copy(pr_h.at[k, pl.ds(base, _SH)], dst)
    pltpu.sync_copy(sc_h.at[c, pl.ds(base, _SH)], scv)

    def _dec(i, carry):
        sl = pl.ds(i * 16, 16)
        ll0 = l0[sl]
        ll1 = l1[sl]
        ll2 = l2[sl]
        ll3 = l3[sl]
        pcx = p0[sl]
        pcy = p1[sl]
        pw = p2[sl]
        ph = p3[sl]
        cx = pcx + (ll0 * f32(0.1)) * pw
        cy = pcy + (ll1 * f32(0.1)) * ph
        w = pw * jnp.exp(ll2 * f32(0.2))
        h = ph * jnp.exp(ll3 * f32(0.2))
        x1 = cx + (-(w / f32(2.0)))
        y1 = cy + (-(h / f32(2.0)))
        x2 = w + x1
        y2 = h + y1
        x1r[sl] = x1
        y1r[sl] = y1
        x2r[sl] = x2
        y2r[sl] = y2
        arr[sl] = (x2 - x1) * (y2 - y1)
        v = scv[sl]
        scv[sl] = jnp.where(v > f32(CONF_THRESH), v, f32(-1.0))
        return carry

    lax.fori_loop(0, _CH, _dec, 0)

    def _gsum(acc, slot):
        vtmp_f[...] = acc.astype(f32)
        pltpu.sync_copy(vtmp_f, arena.at[_BIS0 + slot * _NS + s])
        plsc.subcore_barrier()
        pltpu.sync_copy(arena.at[pl.ds(_BIS0 + slot * _NS, _NS)], cnt_all)
        tot = jnp.zeros((16,), f32)
        for r in range(_NS):
            tot = tot + cnt_all[r]
        return tot

    def _cnt_gt(thr):
        def cb(i, acc):
            v = scv[pl.ds(i * 16, 16)]
            return acc + plsc.all_reduce_population_count(v > thr)
        return lax.fori_loop(0, _CH, cb, jnp.zeros((16,), i32))

    def _bs_val(it, lohi):
        lo, hi = lohi
        mid = lo + (hi - lo) * f32(0.5)
        tot = _gsum(_cnt_gt(mid), it & 1)
        pred = tot < f32(NMS_TOP_K)
        return (jnp.where(pred, lo, mid), jnp.where(pred, mid, hi))

    zf = jnp.zeros((16,), f32)
    _, tval = lax.fori_loop(0, 48, _bs_val, (zf, zf + f32(1.0)))
    c_gt = _gsum(_cnt_gt(tval), 1)
    r_take = f32(NMS_TOP_K) - c_gt

    def _cnt_idx(bnd):
        def cb(i, acc):
            v = scv[pl.ds(i * 16, 16)]
            gidx = base + i * 16 + i16
            hit = (v == tval) & (gidx >= bnd)
            return acc + plsc.all_reduce_population_count(hit)
        return lax.fori_loop(0, _CH, cb, jnp.zeros((16,), i32))

    def _bs_idx(it, lohi):
        lo, hi = lohi
        mid = lo + (hi - lo) // 2
        tot = _gsum(_cnt_idx(mid), it & 1)
        pred = tot <= r_take
        return (jnp.where(pred, lo, mid + 1), jnp.where(pred, mid, hi))

    zi = jnp.zeros((16,), i32)
    _, bnd = lax.fori_loop(0, 16, _bs_idx, (zi, zi + i32(_N)))

    def _pw(i, carry):
        sl = pl.ds(i * 16, 16)
        v = scv[sl]
        gidx = base + i * 16 + i16
        part = (v > tval) | ((v == tval) & (gidx >= bnd))
        scv[sl] = jnp.where(part, v, f32(-1.0))
        return carry

    lax.fori_loop(0, _CH, _pw, 0)

    def _cond(st):
        return st[2]

    def _body(st):
        t, it, _ = st

        def _sel(i, carry):
            bsc, bpos = carry
            v = scv[pl.ds(i * 16, 16)]
            pos = i * 16 + i16
            better = v >= bsc
            return (jnp.where(better, v, bsc), jnp.where(better, pos, bpos))

        bsc, bpos = lax.fori_loop(
            0, _CH, _sel,
            (jnp.full((16,), -2.0, f32), jnp.zeros((16,), i32)))
        m = f32(-2.0)
        jloc = i32(0)
        for r in range(16):
            vr = bsc[r]
            pr = bpos[r]
            better = (vr > m) | ((vr == m) & (pr > jloc))
            m = jnp.where(better, vr, m)
            jloc = jnp.where(better, pr, jloc)
        jv = jnp.zeros((16,), i32) + jloc
        gx1 = plsc.load_gather(x1r, [jv])
        gy1 = plsc.load_gather(y1r, [jv])
        gx2 = plsc.load_gather(x2r, [jv])
        gy2 = plsc.load_gather(y2r, [jv])
        gar = plsc.load_gather(arr, [jv])
        rec = jnp.where(i16 == 0, m,
              jnp.where(i16 == 1, gx1,
              jnp.where(i16 == 2, gy1,
              jnp.where(i16 == 3, gx2,
              jnp.where(i16 == 4, gy2,
              jnp.where(i16 == 5, gar, f32(0.0)))))))
        vtmp_f[...] = rec
        pltpu.sync_copy(vtmp_f, arena.at[_WIN0 + (it & 1) * _NS + s])
        plsc.subcore_barrier()
        pltpu.sync_copy(arena.at[pl.ds(_WIN0 + (it & 1) * _NS, _NS)], wall_f)
        m2 = f32(-2.0)
        w = i32(0)
        for r in range(16):
            vr = wall_f[r][0]
            better = vr >= m2
            m2 = jnp.where(better, vr, m2)
            w = jnp.where(better, i32(r), w)
        picked = m2 > f32(0.0)
        wv = jnp.zeros((16,), i32) + w
        wrow = plsc.load_gather(wall_f, [wv, i16])
        px1 = wrow[1]
        py1 = wrow[2]
        px2 = wrow[3]
        py2 = wrow[4]
        par = wrow[5]

        @pl.when(picked)
        def _():
            rows_loc[t] = jnp.where(i16 == 0, m2,
                          jnp.where(i16 == 1, px1,
                          jnp.where(i16 == 2, py1,
                          jnp.where(i16 == 3, px2,
                          jnp.where(i16 == 4, py2, f32(0.0))))))

        @pl.when(picked)
        def _():
            def _sup(i, carry):
                sl = pl.ds(i * 16, 16)
                v = scv[sl]
                xx1 = jnp.maximum(x1r[sl], px1)
                yy1 = jnp.maximum(y1r[sl], py1)
                xx2 = jnp.minimum(x2r[sl], px2)
                yy2 = jnp.minimum(y2r[sl], py2)
                ww = jnp.maximum(xx2 - xx1, f32(0.0))
                hh = jnp.maximum(yy2 - yy1, f32(0.0))
                inter = ww * hh
                union = arr[sl] - inter + par
                iou = inter / union
                scv[sl] = jnp.where(iou <= f32(NMS_THRESH), v, f32(-1.0))
                return carry

            lax.fori_loop(0, _CH, _sup, 0)

        t1 = jnp.where(picked, t + 1, t)
        go = picked & (t1 < TOP_K)
        return (t1, it + 1, go)

    lax.while_loop(_cond, _body, (i32(0), i32(0), jnp.bool_(True)))

    pltpu.sync_copy(rows_loc.at[pl.ds(s * _RPT, _RPT)],
                    out_h.at[c, pl.ds(s * _RPT, _RPT)])


@jax.jit
def kernel(loc_data, conf_data, prior_data):
    num = loc_data.shape[0]
    pad = _N - NUM_PRIORS
    loc_r = jnp.pad(loc_data.transpose(0, 2, 1), ((0, 0), (0, 0), (0, pad)))
    sc_r = jnp.pad(
        conf_data.reshape(num, NUM_PRIORS, NUM_CLASSES)[:, :, 1],
        ((0, 0), (0, pad)))
    pr_r = jnp.pad(prior_data.T, ((0, 0), (0, pad)))

    mesh = plsc.VectorSubcoreMesh(core_axis_name="c", subcore_axis_name="s")
    f32 = jnp.float32
    run = functools.partial(
        pl.kernel,
        mesh=mesh,
        compiler_params=pltpu.CompilerParams(needs_layout_passes=False),
        out_type=jax.ShapeDtypeStruct((num, _ROWS_PAD, 16), f32),
        scratch_types=[
            pltpu.VMEM((_SH,), f32), pltpu.VMEM((_SH,), f32),
            pltpu.VMEM((_SH,), f32), pltpu.VMEM((_SH,), f32),
            pltpu.VMEM((_SH,), f32), pltpu.VMEM((_SH,), f32),
            pltpu.VMEM((_SH,), f32), pltpu.VMEM((_SH,), f32),
            pltpu.VMEM((_SH,), f32), pltpu.VMEM((_SH,), f32),
            pltpu.VMEM((_SH,), f32), pltpu.VMEM((_SH,), f32),
            pltpu.VMEM((_SH,), f32), pltpu.VMEM((_SH,), f32),
            pltpu.VMEM((16, 16), f32),
            pltpu.VMEM((16, 16), f32),
            pltpu.VMEM((16,), f32),
            pltpu.VMEM((_ROWS_PAD, 16), f32),
            pltpu.VMEM_SHARED((_ARENA_ROWS, 16), f32),
        ],
    )(_sc_body)
    rows = run(loc_r, sc_r, pr_r)

    out = jnp.zeros((num, NUM_CLASSES, TOP_K, 5), jnp.float32)
    return out.at[:, 1, :, :].set(rows[:, :TOP_K, :5])


# speedup vs baseline: 1.0733x; 1.0733x over previous
import functools

import jax
import jax.numpy as jnp
from jax import lax
from jax.experimental import pallas as pl
from jax.experimental.pallas import tpu as pltpu

NUM_CLASSES = 2
TOP_K = 750
NMS_THRESH = 0.3
CONF_THRESH = 0.05
NMS_TOP_K = 5000
NUM_PRIORS = 20000

_ROWS = 160
_PAD_N = _ROWS * 128


def _prep_batch(loc_ref, sc_ref, pr_ref, b, idx):
    f32 = jnp.float32
    l0 = loc_ref[b, 0]
    l1 = loc_ref[b, 1]
    l2 = loc_ref[b, 2]
    l3 = loc_ref[b, 3]
    pcx = pr_ref[0]
    pcy = pr_ref[1]
    pw = pr_ref[2]
    ph = pr_ref[3]
    cx = pcx + (l0 * f32(0.1)) * pw
    cy = pcy + (l1 * f32(0.1)) * ph
    w = pw * jnp.exp(l2 * f32(0.2))
    h = ph * jnp.exp(l3 * f32(0.2))
    x1 = cx + (-(w / f32(2.0)))
    y1 = cy + (-(h / f32(2.0)))
    x2 = w + x1
    y2 = h + y1
    area = (x2 - x1) * (y2 - y1)

    scores = sc_ref[b]
    valid = (scores > f32(CONF_THRESH)) & (idx < NUM_PRIORS)

    sbits = lax.bitcast_convert_type(scores, jnp.int32)
    sb = jnp.where(valid, sbits, jnp.int32(-1))

    def _bs_bits(_, lohi):
        lo, hi = lohi
        mid = (lo + hi) // 2
        cnt = jnp.sum(jnp.where(sb > mid, jnp.int32(1), jnp.int32(0)))
        pred = cnt < NMS_TOP_K
        return (jnp.where(pred, lo, mid + 1), jnp.where(pred, mid, hi))

    _, t_bits = lax.fori_loop(0, 31, _bs_bits,
                              (jnp.int32(0), jnp.int32(0x7F800000)))
    c_gt = jnp.sum(jnp.where(sb > t_bits, jnp.int32(1), jnp.int32(0)))
    r = NMS_TOP_K - c_gt
    eq = sb == t_bits

    def _bs_idx(_, lohi):
        lo, hi = lohi
        mid = (lo + hi) // 2
        cnt = jnp.sum(jnp.where(eq & (idx >= mid), jnp.int32(1), jnp.int32(0)))
        pred = cnt <= r
        return (jnp.where(pred, lo, mid + 1), jnp.where(pred, mid, hi))

    _, bound = lax.fori_loop(0, 16, _bs_idx, (jnp.int32(0), jnp.int32(_PAD_N)))
    participate = valid & ((sb > t_bits) | (eq & (idx >= bound)))
    alive0 = jnp.where(participate, f32(1.0), f32(0.0))
    return scores, x1, y1, x2, y2, area, alive0


def _nms_body(loc_ref, sc_ref, pr_ref, out_ref, a0_ref, a1_ref):
    f32 = jnp.float32
    idx = (lax.broadcasted_iota(jnp.int32, (_ROWS, 128), 0) * 128
           + lax.broadcasted_iota(jnp.int32, (_ROWS, 128), 1))

    data = []
    for b, aref in ((0, a0_ref), (1, a1_ref)):
        scores, x1, y1, x2, y2, area, alive0 = _prep_batch(
            loc_ref, sc_ref, pr_ref, b, idx)
        aref[...] = alive0
        data.append((scores, x1, y1, x2, y2, area, aref,
                     jnp.max(alive0) > f32(0.0)))

    out_ref[...] = jnp.zeros_like(out_ref)
    col5 = lax.broadcasted_iota(jnp.int32, (1, 5), 1)

    def _step(b, t, go_in):
        scores, x1, y1, x2, y2, area, aref, _ = data[b]
        alive = aref[...]
        ms = jnp.where(alive > f32(0.0), scores, f32(-1.0))
        mx = jnp.max(ms)
        j = jnp.max(jnp.where(ms == mx, idx, jnp.int32(-1)))
        pj = idx == j
        pz = f32(0.0)
        px1 = jnp.sum(jnp.where(pj, x1, pz))
        py1 = jnp.sum(jnp.where(pj, y1, pz))
        px2 = jnp.sum(jnp.where(pj, x2, pz))
        py2 = jnp.sum(jnp.where(pj, y2, pz))
        parea = jnp.sum(jnp.where(pj, area, pz))
        xx1 = jnp.maximum(x1, px1)
        yy1 = jnp.maximum(y1, py1)
        xx2 = jnp.minimum(x2, px2)
        yy2 = jnp.minimum(y2, py2)
        ww = jnp.maximum(xx2 - xx1, pz)
        hh = jnp.maximum(yy2 - yy1, pz)
        inter = ww * hh
        union = area - inter + parea
        iou = inter / union
        na = jnp.where((iou <= f32(NMS_THRESH)) & jnp.logical_not(pj),
                       alive, pz)
        aref[...] = jnp.where(go_in, na, alive)

        @pl.when(go_in)
        def _():
            row = jnp.where(col5 == 0, mx,
                  jnp.where(col5 == 1, px1,
                  jnp.where(col5 == 2, py1,
                  jnp.where(col5 == 3, px2, py2))))
            out_ref[b, 1, pl.ds(t, 1), :] = row

        t1 = jnp.where(go_in, t + 1, t)
        go_out = go_in & (jnp.max(na) > f32(0.0)) & (t1 < TOP_K)
        return t1, go_out

    def _cond(st):
        return st[2] | st[3]

    def _body(st):
        t0, t1, g0, g1 = st
        t0n, g0n = _step(0, t0, g0)
        t1n, g1n = _step(1, t1, g1)
        return (t0n, t1n, g0n, g1n)

    lax.while_loop(_cond, _body,
                   (jnp.int32(0), jnp.int32(0), data[0][7], data[1][7]))


@jax.jit
def kernel(loc_data, conf_data, prior_data):
    num = loc_data.shape[0]
    pad = _PAD_N - NUM_PRIORS
    loc_r = jnp.pad(loc_data.transpose(0, 2, 1), ((0, 0), (0, 0), (0, pad)))
    loc_r = loc_r.reshape(num, 4, _ROWS, 128)
    sc = conf_data.reshape(num, NUM_PRIORS, NUM_CLASSES)[:, :, 1]
    sc_r = jnp.pad(sc, ((0, 0), (0, pad))).reshape(num, _ROWS, 128)
    pr_r = jnp.pad(prior_data.T, ((0, 0), (0, pad))).reshape(4, _ROWS, 128)

    out = pl.pallas_call(
        _nms_body,
        out_shape=jax.ShapeDtypeStruct((num, NUM_CLASSES, TOP_K, 5),
                                       jnp.float32),
        scratch_shapes=[pltpu.VMEM((_ROWS, 128), jnp.float32),
                        pltpu.VMEM((_ROWS, 128), jnp.float32)],
    )(loc_r, sc_r, pr_r)
    return out
